# Initial kernel scaffold; baseline (speedup 1.0000x reference)
import jax
import jax.numpy as jnp
from jax.experimental import pallas as pl


def kernel(x, table):
    raise NotImplementedError("write your pallas kernel here")


# speedup vs baseline: 1.9198x; 1.9198x over previous
import jax
import jax.numpy as jnp
from jax.experimental import pallas as pl


_BLOCK_S = 1024


def _copy_kernel(table_ref, out_ref):
    out_ref[0] = table_ref[...]


def kernel(x, table):
    B, S, D = x.shape
    grid = (S // _BLOCK_S, B)
    return pl.pallas_call(
        _copy_kernel,
        grid=grid,
        in_specs=[
            pl.BlockSpec((_BLOCK_S, D), lambda s, b: (s, 0)),
        ],
        out_specs=pl.BlockSpec((1, _BLOCK_S, D), lambda s, b: (b, s, 0)),
        out_shape=jax.ShapeDtypeStruct((B, S, D), table.dtype),
    )(table[:S])


# speedup vs baseline: 2.0800x; 1.0834x over previous
import jax
import jax.numpy as jnp
from jax.experimental import pallas as pl


_BLOCK_S = 2048


def _copy_kernel(table_ref, out_ref):
    out_ref[0] = table_ref[...]


def kernel(x, table):
    B, S, D = x.shape
    grid = (S // _BLOCK_S, B)
    return pl.pallas_call(
        _copy_kernel,
        grid=grid,
        in_specs=[
            pl.BlockSpec((_BLOCK_S, D), lambda s, b: (s, 0)),
        ],
        out_specs=pl.BlockSpec((1, _BLOCK_S, D), lambda s, b: (b, s, 0)),
        out_shape=jax.ShapeDtypeStruct((B, S, D), table.dtype),
    )(table[:S])


# speedup vs baseline: 2.3556x; 1.1325x over previous
import jax
import jax.numpy as jnp
from jax.experimental import pallas as pl


_BLOCK_S = 1024


def _copy_kernel(table_ref, out_ref):
    out_ref[...] = jnp.broadcast_to(table_ref[...][None], out_ref.shape)


def kernel(x, table):
    B, S, D = x.shape
    grid = (S // _BLOCK_S,)
    return pl.pallas_call(
        _copy_kernel,
        grid=grid,
        in_specs=[
            pl.BlockSpec((_BLOCK_S, D), lambda s: (s, 0)),
        ],
        out_specs=pl.BlockSpec((B, _BLOCK_S, D), lambda s: (0, s, 0)),
        out_shape=jax.ShapeDtypeStruct((B, S, D), table.dtype),
    )(table[:S])
